# baseline (speedup 1.0000x reference)
import jax
import jax.numpy as jnp
from jax.experimental import pallas as pl

N = 10000
EPS = 1e-5


def _gcn_conv(x, edge_index, edge_weight, W, b, n_nodes):
    h = x @ W
    row = edge_index[0]
    col = edge_index[1]
    loop = jnp.arange(n_nodes, dtype=edge_index.dtype)
    row = jnp.concatenate([row, loop])
    col = jnp.concatenate([col, loop])
    ew = jnp.concatenate([edge_weight, jnp.ones((n_nodes,), dtype=edge_weight.dtype)])
    deg = jnp.zeros((n_nodes,), dtype=ew.dtype).at[col].add(ew)
    dinv = jnp.where(deg > 0, deg ** -0.5, 0.0)
    norm = dinv[row] * ew * dinv[col]
    out = jnp.zeros_like(h).at[col].add(norm[:, None] * h[row])
    return out + b


def _batch_norm(h, gamma, beta):
    mean = jnp.mean(h, axis=0)
    var = jnp.mean((h - mean) ** 2, axis=0)
    return (h - mean) / jnp.sqrt(var + EPS) * gamma + beta


def _head_kernel(emb_ref, wh1_ref, bh1_ref, wh2_ref, bh2_ref, s_ref):
    s = jnp.maximum(emb_ref[...] @ wh1_ref[...] + bh1_ref[...], 0.0)
    s_ref[...] = s @ wh2_ref[...] + bh2_ref[...]


def kernel(x, edge_index, edge_attr, dist_row_sum, degree, W1, b1, W2, b2, g1, be1, g2, be2, Wd, bd, Wdeg, bdeg, Wm, bm, Wh1, bh1, Wh2, bh2):
    edge_weight = jnp.squeeze(edge_attr)
    h = _gcn_conv(x, edge_index, edge_weight, W1, b1, N)
    h = jax.nn.relu(_batch_norm(h, g1, be1))
    h = _gcn_conv(h, edge_index, edge_weight, W2, b2, N)
    h = jax.nn.relu(_batch_norm(h, g2, be2))
    d_feat = jax.nn.relu(dist_row_sum @ Wd + bd)
    deg_feat = jax.nn.relu(degree @ Wdeg + bdeg)
    h = jnp.concatenate([h, d_feat, deg_feat], axis=1)
    emb = h @ Wm + bm
    s = pl.pallas_call(
        _head_kernel,
        out_shape=jax.ShapeDtypeStruct((N, 1), jnp.float32),
    )(emb, Wh1, bh1, Wh2, bh2)
    score = jnp.squeeze(s)
    return (emb, score)


# speedup vs baseline: 19.3183x; 19.3183x over previous
import functools

import jax
import jax.numpy as jnp
from jax import lax
from jax.experimental import pallas as pl
from jax.experimental.pallas import tpu as pltpu
from jax.experimental.pallas import tpu_sc as plsc

N = 10000
E = 320000
EPS = 1e-5

_NC, _NS, _L = 2, 16, 16
_NW = _NC * _NS
_CH = 80
_EPW = _CH * 128
_EPAD = _NW * _EPW
_NP = 10240
_D = 128
_BLK = 16


def _sc_deg_body(col_hbm, ew_hbm, out_hbm, idx_v, val_v, zb, sh):
    cid = lax.axis_index("c")
    sid = lax.axis_index("s")
    w = cid * _NS + sid

    @pl.loop(0, 40)
    def _zero(i):
        zb[pl.ds(i * 16, 16)] = jnp.zeros((16,), jnp.float32)

    pltpu.sync_copy(zb, sh.at[pl.ds(sid * 640, 640)])
    pltpu.sync_copy(col_hbm.at[w], idx_v)
    pltpu.sync_copy(ew_hbm.at[w], val_v)
    plsc.subcore_barrier()

    @pl.loop(0, _CH)
    def _scat(j):
        pltpu.sync_copy(val_v.at[j], sh.at[idx_v.at[j]], add=True)

    plsc.subcore_barrier()

    @pl.when(sid == 0)
    def _out():
        pltpu.sync_copy(sh, out_hbm.at[cid])


def _sc_deg(col3, ew3):
    return pl.kernel(
        _sc_deg_body,
        out_type=jax.ShapeDtypeStruct((_NC, _NP), jnp.float32),
        mesh=plsc.VectorSubcoreMesh(core_axis_name="c", subcore_axis_name="s"),
        scratch_types=[
            pltpu.VMEM((_CH, 128), jnp.int32),
            pltpu.VMEM((_CH, 128), jnp.float32),
            pltpu.VMEM((640,), jnp.float32),
            pltpu.VMEM_SHARED((_NP,), jnp.float32),
        ],
    )(col3, ew3)


def _splat(v16, e):
    idx = jnp.full((16, 1), e, jnp.int32)
    return lax.gather(
        v16, idx,
        lax.GatherDimensionNumbers(
            offset_dims=(), collapsed_slice_dims=(0,), start_index_map=(0,)),
        slice_sizes=(1,),
        mode=lax.GatherScatterMode.PROMISE_IN_BOUNDS,
    )


def _sc_scatter_body(src, row3, col3, ew3, out_hbm, rowb, colb, ewb, R, sh, sem):
    cid = lax.axis_index("c")
    sid = lax.axis_index("s")
    w = cid * _NS + sid

    @pl.loop(0, 128)
    def _z(r):
        for f in range(8):
            R[r, pl.ds(f * 16, 16)] = jnp.zeros((16,), jnp.float32)

    for k in range(5):
        pltpu.sync_copy(R, sh.at[pl.ds(sid * 640 + k * 128, 128)])

    plsc.subcore_barrier()

    @pl.loop(0, _CH)
    def _chunk(j):
        b = j // _BLK
        jj = j - b * _BLK

        @pl.when(jj == 0)
        def _stage():
            pltpu.sync_copy(row3.at[w, pl.ds(b * _BLK, _BLK)], rowb)
            pltpu.sync_copy(col3.at[w, pl.ds(b * _BLK, _BLK)], colb)
            pltpu.sync_copy(ew3.at[w, pl.ds(b * _BLK, _BLK)], ewb)

        pltpu.async_copy(src.at[rowb.at[jj]], R, sem).wait()

        for g in range(8):
            ew16 = ewb[jj, pl.ds(g * 16, 16)]
            for e in range(16):
                wspl = _splat(ew16, e)
                r = g * 16 + e
                for f in range(8):
                    R[r, pl.ds(f * 16, 16)] = R[r, pl.ds(f * 16, 16)] * wspl

        pltpu.sync_copy(R, sh.at[colb.at[jj]], add=True)

    plsc.subcore_barrier()
    pltpu.sync_copy(sh.at[pl.ds(sid * 640, 640)], out_hbm.at[cid, pl.ds(sid * 640, 640)])


def _sc_scatter(src, row3, col3, ew3):
    outp = pl.kernel(
        _sc_scatter_body,
        out_type=jax.ShapeDtypeStruct((_NC, _NP, _D), jnp.float32),
        mesh=plsc.VectorSubcoreMesh(core_axis_name="c", subcore_axis_name="s"),
        scratch_types=[
            pltpu.VMEM((_BLK, 128), jnp.int32),
            pltpu.VMEM((_BLK, 128), jnp.int32),
            pltpu.VMEM((_BLK, 128), jnp.float32),
            pltpu.VMEM((128, _D), jnp.float32),
            pltpu.VMEM_SHARED((_NP, _D), jnp.float32),
            pltpu.SemaphoreType.DMA,
        ],
    )(src, row3, col3, ew3)
    return outp[0, :N] + outp[1, :N]


def _batch_norm(h, gamma, beta):
    mean = jnp.mean(h, axis=0)
    var = jnp.mean((h - mean) ** 2, axis=0)
    return (h - mean) / jnp.sqrt(var + EPS) * gamma + beta


def kernel(x, edge_index, edge_attr, dist_row_sum, degree, W1, b1, W2, b2, g1, be1, g2, be2, Wd, bd, Wdeg, bdeg, Wm, bm, Wh1, bh1, Wh2, bh2):
    ew = jnp.squeeze(edge_attr)
    row = edge_index[0]
    col = edge_index[1]

    npad = _EPAD - E
    pad_idx = (jnp.arange(npad, dtype=jnp.int32) * 37) % N
    col_p = jnp.concatenate([col, pad_idx]).reshape(_NW, _CH, 128)
    row_p = jnp.concatenate([row, pad_idx]).reshape(_NW, _CH, 128)
    ew_p = jnp.concatenate([ew, jnp.zeros((npad,), jnp.float32)]).reshape(_NW, _CH, 128)

    degp = _sc_deg(col_p, ew_p)
    deg = degp[0, :N] + degp[1, :N] + 1.0
    dinv = jnp.where(deg > 0, lax.rsqrt(deg), 0.0)

    xs = dinv[:, None] * x
    t1 = _sc_scatter(xs, row_p, col_p, ew_p)
    agg1 = dinv[:, None] * (t1 + xs)
    h1 = jax.nn.relu(_batch_norm(
        jnp.dot(agg1, W1, precision=lax.Precision.HIGHEST) + b1, g1, be1))

    p = h1 @ W2
    ps = dinv[:, None] * p
    t2 = _sc_scatter(ps, row_p, col_p, ew_p)
    agg2 = dinv[:, None] * (t2 + ps)
    h2 = jax.nn.relu(_batch_norm(agg2 + b2, g2, be2))

    d_feat = jax.nn.relu(dist_row_sum @ Wd + bd)
    deg_feat = jax.nn.relu(degree @ Wdeg + bdeg)
    h = jnp.concatenate([h2, d_feat, deg_feat], axis=1)
    emb = h @ Wm + bm
    s = jax.nn.relu(emb @ Wh1 + bh1)
    score = jnp.squeeze(s @ Wh2 + bh2)
    return (emb, score)
